# Initial kernel scaffold; baseline (speedup 1.0000x reference)
import jax
import jax.numpy as jnp
from jax.experimental import pallas as pl


def kernel(x, table, hand_W, hand_b, deck_W, deck_b, fc1_W, fc1_b, fc2_W, fc2_b, out_W, out_b):
    raise NotImplementedError("write your pallas kernel here")


# speedup vs baseline: 4.4122x; 4.4122x over previous
import functools

import jax
import jax.numpy as jnp
from jax import lax
from jax.experimental import pallas as pl
from jax.experimental.pallas import tpu as pltpu
from jax.experimental.pallas import tpu_sc as plsc

VOCAB = 65536
EMBED = 32
BATCH = 16384
MAX_HAND = 7
MAX_DECK = 60

NC = 2
NS = 16
NW = NC * NS
BPW = BATCH // NW
BLK = 16
NBLK = BPW // BLK
IDS = 72
CHUNK = 128
CPB = (BLK * IDS) // CHUNK
NCH = (BPW * IDS) // CHUNK


def _sc_pool(table, idx3):
  mesh = plsc.VectorSubcoreMesh(core_axis_name="c", subcore_axis_name="s")

  @functools.partial(
      pl.kernel,
      out_type=[
          jax.ShapeDtypeStruct((BATCH, EMBED), jnp.float32),
          jax.ShapeDtypeStruct((BATCH, EMBED), jnp.float32),
      ],
      mesh=mesh,
      compiler_params=pltpu.CompilerParams(use_tc_tiling_on_sc=False),
      scratch_types=[
          pltpu.VMEM((NCH, CHUNK), jnp.int32),
          pltpu.VMEM((BLK * IDS, EMBED), jnp.float32),
          pltpu.VMEM((BLK, EMBED), jnp.float32),
          pltpu.VMEM((BLK, EMBED), jnp.float32),
          pltpu.SemaphoreType.DMA,
          pltpu.SemaphoreType.DMA,
      ],
  )
  def sc_kernel(table_hbm, idx_hbm, hand_hbm, deck_hbm,
                idx_v, rows_v, hbuf, dbuf, gsem, osem):
    wid = lax.axis_index("s") * NC + lax.axis_index("c")
    pltpu.sync_copy(idx_hbm.at[wid], idx_v)

    def step(b, carry):
      copies = []
      for c in range(CPB):
        copies.append(pltpu.async_copy(
            table_hbm.at[idx_v.at[b * CPB + c]],
            rows_v.at[pl.ds(c * CHUNK, CHUNK)],
            gsem))
      for cp in copies:
        cp.wait()

      def row(r, carry2):
        base = r * IDS
        h0 = rows_v[base, 0:16]
        h1 = rows_v[base, 16:32]
        for k in range(1, 8):
          h0 = h0 + rows_v[base + k, 0:16]
          h1 = h1 + rows_v[base + k, 16:32]
        d0 = rows_v[base + 8, 0:16]
        d1 = rows_v[base + 8, 16:32]
        for k in range(9, IDS):
          d0 = d0 + rows_v[base + k, 0:16]
          d1 = d1 + rows_v[base + k, 16:32]
        hbuf[r, 0:16] = h0
        hbuf[r, 16:32] = h1
        dbuf[r, 0:16] = d0
        dbuf[r, 16:32] = d1
        return carry2

      lax.fori_loop(0, BLK, row, 0)
      out_base = wid * BPW + b * BLK
      cp_h = pltpu.async_copy(hbuf, hand_hbm.at[pl.ds(out_base, BLK)], osem)
      cp_d = pltpu.async_copy(dbuf, deck_hbm.at[pl.ds(out_base, BLK)], osem)
      cp_h.wait()
      cp_d.wait()
      return carry

    lax.fori_loop(0, NBLK, step, 0)

  return sc_kernel(table, idx3)


def _tc_mlp_body(x_ref, hs_ref, ds_ref, hWt_ref, hb_ref, dWt_ref, db_ref,
                 f1m_ref, f1h_ref, f1d_ref, f1b_ref, f2t_ref, f2b_ref,
                 oWt_ref, ob_ref, out_ref):
  xb = x_ref[...]
  mull = xb[:, 0:1]
  cnt_h = jnp.sum((xb[:, 1:1 + MAX_HAND] != 0.0).astype(jnp.float32),
                  axis=1, keepdims=True)
  cnt_d = jnp.sum((xb[:, 1 + MAX_HAND:1 + MAX_HAND + MAX_DECK] != 0.0)
                  .astype(jnp.float32), axis=1, keepdims=True)
  hp = hs_ref[...] / (cnt_h + 1e-8)
  dp = ds_ref[...] / (cnt_d + 1e-8)
  dot = functools.partial(jnp.dot, preferred_element_type=jnp.float32,
                          precision=jax.lax.Precision.HIGHEST)
  hf = jnp.maximum(dot(hp, hWt_ref[...]) + hb_ref[...], 0.0)
  df = jnp.maximum(dot(dp, dWt_ref[...]) + db_ref[...], 0.0)
  h1 = dot(hf, f1h_ref[...]) + dot(df, f1d_ref[...])
  h1 = jnp.maximum(h1 + mull * f1m_ref[...] + f1b_ref[...], 0.0)
  h2 = jnp.maximum(dot(h1, f2t_ref[...]) + f2b_ref[...], 0.0)
  out_ref[...] = dot(h2, oWt_ref[...]) + ob_ref[...]


def _tc_mlp(x, hand_sum, deck_sum, hWt, hb, dWt, db, f1m, f1h, f1d, f1b,
            f2t, f2b, oWt, ob, interpret=False):
  bb = 2048
  grid = (BATCH // bb,)
  full = lambda a: pl.BlockSpec(a.shape, lambda i: (0,) * a.ndim)
  return pl.pallas_call(
      _tc_mlp_body,
      grid=grid,
      in_specs=[
          pl.BlockSpec((bb, x.shape[1]), lambda i: (i, 0)),
          pl.BlockSpec((bb, EMBED), lambda i: (i, 0)),
          pl.BlockSpec((bb, EMBED), lambda i: (i, 0)),
          full(hWt), full(hb), full(dWt), full(db),
          full(f1m), full(f1h), full(f1d), full(f1b),
          full(f2t), full(f2b), full(oWt), full(ob),
      ],
      out_specs=pl.BlockSpec((bb, 2), lambda i: (i, 0)),
      out_shape=jax.ShapeDtypeStruct((BATCH, 2), jnp.float32),
      interpret=interpret,
  )(x, hand_sum, deck_sum, hWt, hb, dWt, db, f1m, f1h, f1d, f1b,
    f2t, f2b, oWt, ob)


def kernel(x, table, hand_W, hand_b, deck_W, deck_b, fc1_W, fc1_b,
           fc2_W, fc2_b, out_W, out_b):
  ids = x[:, 1:].astype(jnp.int32)
  hand_ids = ids[:, :MAX_HAND]
  deck_ids = ids[:, MAX_HAND:]
  pad_h = jnp.zeros((BATCH, 8 - MAX_HAND), jnp.int32)
  pad_d = jnp.zeros((BATCH, 64 - MAX_DECK), jnp.int32)
  ids72 = jnp.concatenate([hand_ids, pad_h, deck_ids, pad_d], axis=1)
  idx3 = ids72.reshape(NW, NCH, CHUNK)

  hand_sum, deck_sum = _sc_pool(table, idx3)

  hWt = hand_W.T
  dWt = deck_W.T
  f1m = fc1_W[:, 0:1].T
  f1h = fc1_W[:, 1:1 + 32].T
  f1d = fc1_W[:, 1 + 32:1 + 64].T
  f2t = fc2_W.T
  oWt = out_W.T
  return _tc_mlp(x, hand_sum, deck_sum,
                 hWt, hand_b.reshape(1, 32), dWt, deck_b.reshape(1, 32),
                 f1m, f1h, f1d, fc1_b.reshape(1, 64),
                 f2t, fc2_b.reshape(1, 32), oWt, out_b.reshape(1, 2))


# speedup vs baseline: 4.4598x; 1.0108x over previous
import functools

import jax
import jax.numpy as jnp
from jax import lax
from jax.experimental import pallas as pl
from jax.experimental.pallas import tpu as pltpu
from jax.experimental.pallas import tpu_sc as plsc

VOCAB = 65536
EMBED = 32
BATCH = 16384
MAX_HAND = 7
MAX_DECK = 60

NC = 2
NS = 16
NW = NC * NS
BPW = BATCH // NW
BLK = 16
NBLK = BPW // BLK
IDS = 72
CHUNK = 128
CPB = (BLK * IDS) // CHUNK
NCH = (BPW * IDS) // CHUNK


NQ = BPW // CHUNK


def _sc_pool(table, idx3, zeros):
  mesh = plsc.VectorSubcoreMesh(core_axis_name="c", subcore_axis_name="s")

  @functools.partial(
      pl.kernel,
      out_type=[
          jax.ShapeDtypeStruct((BATCH, EMBED), jnp.float32),
          jax.ShapeDtypeStruct((BATCH, EMBED), jnp.float32),
      ],
      mesh=mesh,
      compiler_params=pltpu.CompilerParams(use_tc_tiling_on_sc=False),
      scratch_types=[
          pltpu.VMEM((NCH, CHUNK), jnp.int32),
          pltpu.VMEM((BPW, EMBED), jnp.float32),
          pltpu.VMEM((BPW, EMBED), jnp.float32),
          pltpu.SemaphoreType.DMA,
          pltpu.SemaphoreType.DMA,
          pltpu.SemaphoreType.DMA,
      ],
  )
  def sc_kernel(table_hbm, idx_hbm, zeros_hbm, hand_hbm, deck_hbm,
                idx_v, hacc, dacc, zsem, gsem, osem):
    wid = lax.axis_index("s") * NC + lax.axis_index("c")
    pltpu.sync_copy(idx_hbm.at[wid], idx_v)
    zcp = []
    for q in range(NQ):
      zcp.append(pltpu.async_copy(
          zeros_hbm, hacc.at[pl.ds(q * CHUNK, CHUNK)], zsem))
      zcp.append(pltpu.async_copy(
          zeros_hbm, dacc.at[pl.ds(q * CHUNK, CHUNK)], zsem))
    for cp in zcp:
      cp.wait()

    for q in range(NQ):
      hdst = hacc.at[pl.ds(q * CHUNK, CHUNK)]
      ddst = dacc.at[pl.ds(q * CHUNK, CHUNK)]

      def hand_stream(j, carry, q=q, hdst=hdst):
        pltpu.async_copy(table_hbm.at[idx_v.at[q * IDS + j]], hdst, gsem,
                         add=True)
        return carry

      def deck_stream(j, carry, q=q, ddst=ddst):
        pltpu.async_copy(table_hbm.at[idx_v.at[q * IDS + j]], ddst, gsem,
                         add=True)
        return carry

      lax.fori_loop(0, 8, hand_stream, 0)
      lax.fori_loop(8, IDS, deck_stream, 0)

    def drain(i, carry):
      pltpu.make_async_copy(
          table_hbm.at[pl.ds(0, CHUNK)],
          hacc.at[pl.ds(0, CHUNK)], gsem).wait()
      return carry

    lax.fori_loop(0, NCH, drain, 0)

    out_base = wid * BPW
    cp_h = pltpu.async_copy(hacc, hand_hbm.at[pl.ds(out_base, BPW)], osem)
    cp_d = pltpu.async_copy(dacc, deck_hbm.at[pl.ds(out_base, BPW)], osem)
    cp_h.wait()
    cp_d.wait()

  return sc_kernel(table, idx3, zeros)


def _tc_mlp_body(x_ref, hs_ref, ds_ref, hWt_ref, hb_ref, dWt_ref, db_ref,
                 f1m_ref, f1h_ref, f1d_ref, f1b_ref, f2t_ref, f2b_ref,
                 oWt_ref, ob_ref, out_ref):
  xb = x_ref[...]
  mull = xb[:, 0:1]
  cnt_h = jnp.sum((xb[:, 1:1 + MAX_HAND] != 0.0).astype(jnp.float32),
                  axis=1, keepdims=True)
  cnt_d = jnp.sum((xb[:, 1 + MAX_HAND:1 + MAX_HAND + MAX_DECK] != 0.0)
                  .astype(jnp.float32), axis=1, keepdims=True)
  hp = hs_ref[...] / (cnt_h + 1e-8)
  dp = ds_ref[...] / (cnt_d + 1e-8)
  dot = functools.partial(jnp.dot, preferred_element_type=jnp.float32,
                          precision=jax.lax.Precision.HIGHEST)
  hf = jnp.maximum(dot(hp, hWt_ref[...]) + hb_ref[...], 0.0)
  df = jnp.maximum(dot(dp, dWt_ref[...]) + db_ref[...], 0.0)
  h1 = dot(hf, f1h_ref[...]) + dot(df, f1d_ref[...])
  h1 = jnp.maximum(h1 + mull * f1m_ref[...] + f1b_ref[...], 0.0)
  h2 = jnp.maximum(dot(h1, f2t_ref[...]) + f2b_ref[...], 0.0)
  out_ref[...] = dot(h2, oWt_ref[...]) + ob_ref[...]


def _tc_mlp(x, hand_sum, deck_sum, hWt, hb, dWt, db, f1m, f1h, f1d, f1b,
            f2t, f2b, oWt, ob, interpret=False):
  bb = 2048
  grid = (BATCH // bb,)
  full = lambda a: pl.BlockSpec(a.shape, lambda i: (0,) * a.ndim)
  return pl.pallas_call(
      _tc_mlp_body,
      grid=grid,
      in_specs=[
          pl.BlockSpec((bb, x.shape[1]), lambda i: (i, 0)),
          pl.BlockSpec((bb, EMBED), lambda i: (i, 0)),
          pl.BlockSpec((bb, EMBED), lambda i: (i, 0)),
          full(hWt), full(hb), full(dWt), full(db),
          full(f1m), full(f1h), full(f1d), full(f1b),
          full(f2t), full(f2b), full(oWt), full(ob),
      ],
      out_specs=pl.BlockSpec((bb, 2), lambda i: (i, 0)),
      out_shape=jax.ShapeDtypeStruct((BATCH, 2), jnp.float32),
      interpret=interpret,
  )(x, hand_sum, deck_sum, hWt, hb, dWt, db, f1m, f1h, f1d, f1b,
    f2t, f2b, oWt, ob)


def kernel(x, table, hand_W, hand_b, deck_W, deck_b, fc1_W, fc1_b,
           fc2_W, fc2_b, out_W, out_b):
  ids = x[:, 1:].astype(jnp.int32)
  hand_ids = ids[:, :MAX_HAND]
  deck_ids = ids[:, MAX_HAND:]
  pad_h = jnp.zeros((BATCH, 8 - MAX_HAND), jnp.int32)
  pad_d = jnp.zeros((BATCH, 64 - MAX_DECK), jnp.int32)
  ids72 = jnp.concatenate([hand_ids, pad_h, deck_ids, pad_d], axis=1)
  idx3 = (ids72.reshape(NW, NQ, CHUNK, IDS)
          .transpose(0, 1, 3, 2)
          .reshape(NW, NCH, CHUNK))
  zeros = jnp.zeros((CHUNK, EMBED), jnp.float32)

  hand_sum, deck_sum = _sc_pool(table, idx3, zeros)

  hWt = hand_W.T
  dWt = deck_W.T
  f1m = fc1_W[:, 0:1].T
  f1h = fc1_W[:, 1:1 + 32].T
  f1d = fc1_W[:, 1 + 32:1 + 64].T
  f2t = fc2_W.T
  oWt = out_W.T
  return _tc_mlp(x, hand_sum, deck_sum,
                 hWt, hand_b.reshape(1, 32), dWt, deck_b.reshape(1, 32),
                 f1m, f1h, f1d, fc1_b.reshape(1, 64),
                 f2t, fc2_b.reshape(1, 32), oWt, out_b.reshape(1, 2))


# speedup vs baseline: 19.1534x; 4.2947x over previous
import functools

import jax
import jax.numpy as jnp
from jax import lax
from jax.experimental import pallas as pl
from jax.experimental.pallas import tpu as pltpu
from jax.experimental.pallas import tpu_sc as plsc

VOCAB = 65536
EMBED = 32
BATCH = 16384
MAX_HAND = 7
MAX_DECK = 60
NCOL = MAX_HAND + MAX_DECK

NC = 2
NS = 16
NW = NC * NS
BPW = BATCH // NW
BLK = 16
NBLK = BPW // BLK
CHUNK = 128
NQ0 = BPW // CHUNK
NCH = NQ0 * NCOL


def _sc_pool(table, idx3, zeros):
  mesh = plsc.VectorSubcoreMesh(core_axis_name="c", subcore_axis_name="s")

  @functools.partial(
      pl.kernel,
      out_type=[
          jax.ShapeDtypeStruct((BATCH, EMBED), jnp.float32),
          jax.ShapeDtypeStruct((BATCH, EMBED), jnp.float32),
      ],
      mesh=mesh,
      compiler_params=pltpu.CompilerParams(use_tc_tiling_on_sc=False),
      scratch_types=[
          pltpu.VMEM((NCH, CHUNK), jnp.int32),
          pltpu.VMEM((BPW, EMBED), jnp.float32),
          pltpu.VMEM((BPW, EMBED), jnp.float32),
          pltpu.SemaphoreType.DMA,
          pltpu.SemaphoreType.DMA,
          pltpu.SemaphoreType.DMA,
      ],
  )
  def sc_kernel(table_hbm, idx_hbm, zeros_hbm, hand_hbm, deck_hbm,
                idx_v, hacc, dacc, zsem, gsem, osem):
    wid = lax.axis_index("s") * NC + lax.axis_index("c")
    pltpu.sync_copy(idx_hbm.at[wid], idx_v)
    zcp = []
    for q in range(NQ0):
      zcp.append(pltpu.async_copy(
          zeros_hbm, hacc.at[pl.ds(q * CHUNK, CHUNK)], zsem))
      zcp.append(pltpu.async_copy(
          zeros_hbm, dacc.at[pl.ds(q * CHUNK, CHUNK)], zsem))
    for cp in zcp:
      cp.wait()

    for q in range(NQ0):
      hdst = hacc.at[pl.ds(q * CHUNK, CHUNK)]
      ddst = dacc.at[pl.ds(q * CHUNK, CHUNK)]

      def hand_stream(j, carry, q=q, hdst=hdst):
        pltpu.async_copy(table_hbm.at[idx_v.at[q * NCOL + j]], hdst, gsem,
                         add=True)
        return carry

      def deck_stream(j, carry, q=q, ddst=ddst):
        pltpu.async_copy(table_hbm.at[idx_v.at[q * NCOL + j]], ddst, gsem,
                         add=True)
        return carry

      lax.fori_loop(0, MAX_HAND, hand_stream, 0)
      lax.fori_loop(MAX_HAND, NCOL, deck_stream, 0)

    def drain(i, carry):
      pltpu.make_async_copy(
          table_hbm.at[pl.ds(0, CHUNK)],
          hacc.at[pl.ds(0, CHUNK)], gsem).wait()
      return carry

    lax.fori_loop(0, NCH, drain, 0)

    out_base = wid * BPW
    cp_h = pltpu.async_copy(hacc, hand_hbm.at[pl.ds(out_base, BPW)], osem)
    cp_d = pltpu.async_copy(dacc, deck_hbm.at[pl.ds(out_base, BPW)], osem)
    cp_h.wait()
    cp_d.wait()

  return sc_kernel(table, idx3, zeros)


def _tc_mlp_body(x_ref, hs_ref, ds_ref, hWt_ref, hb_ref, dWt_ref, db_ref,
                 f1m_ref, f1h_ref, f1d_ref, f1b_ref, f2t_ref, f2b_ref,
                 oWt_ref, ob_ref, out_ref):
  xb = x_ref[...]
  mull = xb[:, 0:1]
  cnt_h = jnp.sum((xb[:, 1:1 + MAX_HAND] != 0.0).astype(jnp.float32),
                  axis=1, keepdims=True)
  cnt_d = jnp.sum((xb[:, 1 + MAX_HAND:1 + MAX_HAND + MAX_DECK] != 0.0)
                  .astype(jnp.float32), axis=1, keepdims=True)
  hp = hs_ref[...] / (cnt_h + 1e-8)
  dp = ds_ref[...] / (cnt_d + 1e-8)
  dot = functools.partial(jnp.dot, preferred_element_type=jnp.float32,
                          precision=jax.lax.Precision.HIGHEST)
  hf = jnp.maximum(dot(hp, hWt_ref[...]) + hb_ref[...], 0.0)
  df = jnp.maximum(dot(dp, dWt_ref[...]) + db_ref[...], 0.0)
  h1 = dot(hf, f1h_ref[...]) + dot(df, f1d_ref[...])
  h1 = jnp.maximum(h1 + mull * f1m_ref[...] + f1b_ref[...], 0.0)
  h2 = jnp.maximum(dot(h1, f2t_ref[...]) + f2b_ref[...], 0.0)
  out_ref[...] = dot(h2, oWt_ref[...]) + ob_ref[...]


def _tc_mlp(x, hand_sum, deck_sum, hWt, hb, dWt, db, f1m, f1h, f1d, f1b,
            f2t, f2b, oWt, ob, interpret=False):
  bb = 2048
  grid = (BATCH // bb,)
  full = lambda a: pl.BlockSpec(a.shape, lambda i: (0,) * a.ndim)
  return pl.pallas_call(
      _tc_mlp_body,
      grid=grid,
      in_specs=[
          pl.BlockSpec((bb, x.shape[1]), lambda i: (i, 0)),
          pl.BlockSpec((bb, EMBED), lambda i: (i, 0)),
          pl.BlockSpec((bb, EMBED), lambda i: (i, 0)),
          full(hWt), full(hb), full(dWt), full(db),
          full(f1m), full(f1h), full(f1d), full(f1b),
          full(f2t), full(f2b), full(oWt), full(ob),
      ],
      out_specs=pl.BlockSpec((bb, 2), lambda i: (i, 0)),
      out_shape=jax.ShapeDtypeStruct((BATCH, 2), jnp.float32),
      interpret=interpret,
  )(x, hand_sum, deck_sum, hWt, hb, dWt, db, f1m, f1h, f1d, f1b,
    f2t, f2b, oWt, ob)


def kernel(x, table, hand_W, hand_b, deck_W, deck_b, fc1_W, fc1_b,
           fc2_W, fc2_b, out_W, out_b):
  ids = x[:, 1:].astype(jnp.int32)
  idx3 = (ids.reshape(NW, NQ0, CHUNK, NCOL)
          .transpose(0, 1, 3, 2)
          .reshape(NW, NCH, CHUNK))
  zeros = jnp.zeros((CHUNK, EMBED), jnp.float32)

  hand_sum, deck_sum = _sc_pool(table, idx3, zeros)

  hWt = hand_W.T
  dWt = deck_W.T
  f1m = fc1_W[:, 0:1].T
  f1h = fc1_W[:, 1:1 + 32].T
  f1d = fc1_W[:, 1 + 32:1 + 64].T
  f2t = fc2_W.T
  oWt = out_W.T
  return _tc_mlp(x, hand_sum, deck_sum,
                 hWt, hand_b.reshape(1, 32), dWt, deck_b.reshape(1, 32),
                 f1m, f1h, f1d, fc1_b.reshape(1, 64),
                 f2t, fc2_b.reshape(1, 32), oWt, out_b.reshape(1, 2))


# speedup vs baseline: 21.1079x; 1.1020x over previous
import functools

import jax
import jax.numpy as jnp
from jax import lax
from jax.experimental import pallas as pl
from jax.experimental.pallas import tpu as pltpu
from jax.experimental.pallas import tpu_sc as plsc

VOCAB = 65536
EMBED = 32
BATCH = 16384
MAX_HAND = 7
MAX_DECK = 60
NCOL = MAX_HAND + MAX_DECK

NC = 2
NS = 16
NW = NC * NS
BPW = BATCH // NW
BLK = 16
NBLK = BPW // BLK
CHUNK = 128
NQ0 = BPW // CHUNK
NCH = NQ0 * NCOL


def _sc_pool(table, idx3, zeros):
  mesh = plsc.VectorSubcoreMesh(core_axis_name="c", subcore_axis_name="s")

  @functools.partial(
      pl.kernel,
      out_type=[
          jax.ShapeDtypeStruct((BATCH, EMBED), jnp.float32),
          jax.ShapeDtypeStruct((BATCH, EMBED), jnp.float32),
      ],
      mesh=mesh,
      compiler_params=pltpu.CompilerParams(use_tc_tiling_on_sc=False),
      scratch_types=[
          pltpu.VMEM((NCH, CHUNK), jnp.int32),
          pltpu.VMEM((BPW, EMBED), jnp.float32),
          pltpu.VMEM((BPW, EMBED), jnp.float32),
          pltpu.SemaphoreType.DMA,
          pltpu.SemaphoreType.DMA,
          pltpu.SemaphoreType.DMA,
      ],
  )
  def sc_kernel(table_hbm, idx_hbm, zeros_hbm, hand_hbm, deck_hbm,
                idx_v, hacc, dacc, zsem, gsem, osem):
    wid = lax.axis_index("s") * NC + lax.axis_index("c")
    pltpu.sync_copy(idx_hbm.at[wid], idx_v)
    zcp = []
    for q in range(NQ0):
      zcp.append(pltpu.async_copy(
          zeros_hbm, hacc.at[pl.ds(q * CHUNK, CHUNK)], zsem))
      zcp.append(pltpu.async_copy(
          zeros_hbm, dacc.at[pl.ds(q * CHUNK, CHUNK)], zsem))
    for cp in zcp:
      cp.wait()

    for q in range(NQ0):
      hdst = hacc.at[pl.ds(q * CHUNK, CHUNK)]
      ddst = dacc.at[pl.ds(q * CHUNK, CHUNK)]

      def hand_stream(j, carry, q=q, hdst=hdst):
        pltpu.async_copy(table_hbm.at[idx_v.at[q * NCOL + j]], hdst, gsem,
                         add=True)
        return carry

      def deck_stream(j, carry, q=q, ddst=ddst):
        pltpu.async_copy(table_hbm.at[idx_v.at[q * NCOL + j]], ddst, gsem,
                         add=True)
        return carry

      lax.fori_loop(0, MAX_HAND, hand_stream, 0)
      lax.fori_loop(MAX_HAND, NCOL, deck_stream, 0)

    def drain(i, carry):
      pltpu.make_async_copy(
          table_hbm.at[pl.ds(0, CHUNK)],
          hacc.at[pl.ds(0, CHUNK)], gsem).wait()
      return carry

    lax.fori_loop(0, NCH, drain, 0)

    out_base = wid * BPW
    cp_h = pltpu.async_copy(hacc, hand_hbm.at[pl.ds(out_base, BPW)], osem)
    cp_d = pltpu.async_copy(dacc, deck_hbm.at[pl.ds(out_base, BPW)], osem)
    cp_h.wait()
    cp_d.wait()

  return sc_kernel(table, idx3, zeros)


def _dg(a, w, precision):
  return lax.dot_general(a, w, (((1,), (1,)), ((), ())),
                         precision=precision,
                         preferred_element_type=jnp.float32)


def _tc_mlp_body(x_ref, hs_ref, ds_ref, hW_ref, hb_ref, dW_ref, db_ref,
                 f1W_ref, f1b_ref, f2W_ref, f2b_ref, oW_ref, ob_ref,
                 out_ref):
  hi = jax.lax.Precision.HIGHEST
  lo = jax.lax.Precision.DEFAULT
  xb = x_ref[...]
  mull = xb[:, 0:1]
  cnt_h = jnp.sum((xb[:, 1:1 + MAX_HAND] != 0.0).astype(jnp.float32),
                  axis=1, keepdims=True)
  cnt_d = jnp.sum((xb[:, 1 + MAX_HAND:1 + MAX_HAND + MAX_DECK] != 0.0)
                  .astype(jnp.float32), axis=1, keepdims=True)
  hp = hs_ref[...] / (cnt_h + 1e-8)
  dp = ds_ref[...] / (cnt_d + 1e-8)
  hf = jnp.maximum(_dg(hp, hW_ref[...], lo) + hb_ref[...][None, :], 0.0)
  df = jnp.maximum(_dg(dp, dW_ref[...], lo) + db_ref[...][None, :], 0.0)
  f1W = f1W_ref[...]
  h1 = _dg(jnp.concatenate([hf, df], axis=1), f1W[:, 1:], lo)
  h1 = jnp.maximum(h1 + mull * f1W[:, 0][None, :] + f1b_ref[...][None, :],
                   0.0)
  h2 = jnp.maximum(_dg(h1, f2W_ref[...], hi) + f2b_ref[...][None, :], 0.0)
  out_ref[...] = _dg(h2, oW_ref[...], hi) + ob_ref[...][None, :]


def _tc_mlp(x, hand_sum, deck_sum, hand_W, hand_b, deck_W, deck_b,
            fc1_W, fc1_b, fc2_W, fc2_b, out_W, out_b, interpret=False):
  bb = 2048
  grid = (BATCH // bb,)
  full = lambda a: pl.BlockSpec(a.shape, lambda i: (0,) * a.ndim)
  return pl.pallas_call(
      _tc_mlp_body,
      grid=grid,
      in_specs=[
          pl.BlockSpec((bb, x.shape[1]), lambda i: (i, 0)),
          pl.BlockSpec((bb, EMBED), lambda i: (i, 0)),
          pl.BlockSpec((bb, EMBED), lambda i: (i, 0)),
          full(hand_W), full(hand_b), full(deck_W), full(deck_b),
          full(fc1_W), full(fc1_b), full(fc2_W), full(fc2_b),
          full(out_W), full(out_b),
      ],
      out_specs=pl.BlockSpec((bb, 2), lambda i: (i, 0)),
      out_shape=jax.ShapeDtypeStruct((BATCH, 2), jnp.float32),
      interpret=interpret,
  )(x, hand_sum, deck_sum, hand_W, hand_b, deck_W, deck_b,
    fc1_W, fc1_b, fc2_W, fc2_b, out_W, out_b)


def kernel(x, table, hand_W, hand_b, deck_W, deck_b, fc1_W, fc1_b,
           fc2_W, fc2_b, out_W, out_b):
  ids = x[:, 1:].astype(jnp.int32)
  idx3 = (ids.reshape(NW, NQ0, CHUNK, NCOL)
          .transpose(0, 1, 3, 2)
          .reshape(NW, NCH, CHUNK))
  zeros = jnp.zeros((CHUNK, EMBED), jnp.float32)

  hand_sum, deck_sum = _sc_pool(table, idx3, zeros)

  return _tc_mlp(x, hand_sum, deck_sum, hand_W, hand_b, deck_W, deck_b,
                 fc1_W, fc1_b, fc2_W, fc2_b, out_W, out_b)


# speedup vs baseline: 24.3308x; 1.1527x over previous
import functools

import jax
import jax.numpy as jnp
from jax import lax
from jax.experimental import pallas as pl
from jax.experimental.pallas import tpu as pltpu
from jax.experimental.pallas import tpu_sc as plsc

VOCAB = 65536
EMBED = 32
BATCH = 16384
MAX_HAND = 7
MAX_DECK = 60
NCOL = MAX_HAND + MAX_DECK
XCOL = 1 + NCOL

NC = 2
NS = 16
NW = NC * NS
BPW = BATCH // NW
CHUNK = 128
NQ0 = BPW // CHUNK
NCH = NQ0 * NCOL
LANE = 16


def _sc_pool(table, x):
  mesh = plsc.VectorSubcoreMesh(core_axis_name="c", subcore_axis_name="s")

  @functools.partial(
      pl.kernel,
      out_type=[
          jax.ShapeDtypeStruct((BATCH, EMBED), jnp.float32),
          jax.ShapeDtypeStruct((BATCH, EMBED), jnp.float32),
      ],
      mesh=mesh,
      compiler_params=pltpu.CompilerParams(use_tc_tiling_on_sc=False,
                                           needs_layout_passes=False),
      scratch_types=[
          pltpu.VMEM((BPW, XCOL), jnp.float32),
          pltpu.VMEM((NCH, CHUNK), jnp.int32),
          pltpu.VMEM((BPW, EMBED), jnp.float32),
          pltpu.VMEM((BPW, EMBED), jnp.float32),
          pltpu.SemaphoreType.DMA,
          pltpu.SemaphoreType.DMA,
      ],
  )
  def sc_kernel(table_hbm, x_hbm, hand_hbm, deck_hbm,
                xv, idx_v, hacc, dacc, gsem, osem):
    wid = lax.axis_index("s") * NC + lax.axis_index("c")
    pltpu.sync_copy(x_hbm.at[pl.ds(wid * BPW, BPW)], xv)

    zv = jnp.zeros((LANE,), jnp.float32)

    def zero_row(i, carry):
      hacc[i, 0:LANE] = zv
      hacc[i, LANE:EMBED] = zv
      dacc[i, 0:LANE] = zv
      dacc[i, LANE:EMBED] = zv
      return carry

    lax.fori_loop(0, BPW, zero_row, 0)

    lane = lax.iota(jnp.int32, LANE)

    def build_q(q, carry):
      def build_j(j, carry2):
        colv = jnp.full((LANE,), 0, jnp.int32) + (j + 1)
        for k in range(CHUNK // LANE):
          rowv = lane + (q * CHUNK + k * LANE)
          vals = plsc.load_gather(xv, [rowv, colv])
          idx_v[q * NCOL + j, k * LANE:(k + 1) * LANE] = (
              vals.astype(jnp.int32))
        return carry2

      return lax.fori_loop(0, NCOL, build_j, carry)

    lax.fori_loop(0, NQ0, build_q, 0)

    for q in range(NQ0):
      hdst = hacc.at[pl.ds(q * CHUNK, CHUNK)]
      ddst = dacc.at[pl.ds(q * CHUNK, CHUNK)]

      def hand_stream(j, carry, q=q, hdst=hdst):
        pltpu.async_copy(table_hbm.at[idx_v.at[q * NCOL + j]], hdst, gsem,
                         add=True)
        return carry

      def deck_stream(j, carry, q=q, ddst=ddst):
        pltpu.async_copy(table_hbm.at[idx_v.at[q * NCOL + j]], ddst, gsem,
                         add=True)
        return carry

      lax.fori_loop(0, MAX_HAND, hand_stream, 0)
      lax.fori_loop(MAX_HAND, NCOL, deck_stream, 0)

    def drain(i, carry):
      pltpu.make_async_copy(
          table_hbm.at[pl.ds(0, CHUNK)],
          hacc.at[pl.ds(0, CHUNK)], gsem).wait()
      return carry

    lax.fori_loop(0, NCH, drain, 0)

    out_base = wid * BPW
    cp_h = pltpu.async_copy(hacc, hand_hbm.at[pl.ds(out_base, BPW)], osem)
    cp_d = pltpu.async_copy(dacc, deck_hbm.at[pl.ds(out_base, BPW)], osem)
    cp_h.wait()
    cp_d.wait()

  return sc_kernel(table, x)


def _dg(a, w, precision):
  return lax.dot_general(a, w, (((1,), (1,)), ((), ())),
                         precision=precision,
                         preferred_element_type=jnp.float32)


def _tc_mlp_body(x_ref, hs_ref, ds_ref, sel_ref, hW_ref, hb_ref, dW_ref,
                 db_ref, f1W_ref, f1b_ref, f2W_ref, f2b_ref, oW_ref,
                 ob_ref, out_ref):
  hi = jax.lax.Precision.HIGHEST
  lo = jax.lax.Precision.DEFAULT
  xb = x_ref[...]
  mull = xb[:, 0:1]
  mask = (xb != 0.0).astype(jnp.float32)
  cnt = _dg(mask, sel_ref[...], lo)
  inv = 1.0 / (cnt + 1e-8)
  hp = hs_ref[...] * inv[:, 0:1]
  dp = ds_ref[...] * inv[:, 1:2]
  hf = jnp.maximum(_dg(hp, hW_ref[...], lo) + hb_ref[...][None, :], 0.0)
  df = jnp.maximum(_dg(dp, dW_ref[...], lo) + db_ref[...][None, :], 0.0)
  f1W = f1W_ref[...]
  h1 = _dg(hf, f1W[:, 1:1 + EMBED], lo) + _dg(df, f1W[:, 1 + EMBED:], lo)
  h1 = jnp.maximum(h1 + mull * f1W[:, 0][None, :] + f1b_ref[...][None, :],
                   0.0)
  h2 = jnp.maximum(_dg(h1, f2W_ref[...], hi) + f2b_ref[...][None, :], 0.0)
  out_ref[...] = _dg(h2, oW_ref[...], hi) + ob_ref[...][None, :]


def _tc_mlp(x, hand_sum, deck_sum, sel, hand_W, hand_b, deck_W, deck_b,
            fc1_W, fc1_b, fc2_W, fc2_b, out_W, out_b, interpret=False):
  bb = 4096
  grid = (BATCH // bb,)
  full = lambda a: pl.BlockSpec(a.shape, lambda i: (0,) * a.ndim)
  return pl.pallas_call(
      _tc_mlp_body,
      grid=grid,
      in_specs=[
          pl.BlockSpec((bb, XCOL), lambda i: (i, 0)),
          pl.BlockSpec((bb, EMBED), lambda i: (i, 0)),
          pl.BlockSpec((bb, EMBED), lambda i: (i, 0)),
          full(sel), full(hand_W), full(hand_b), full(deck_W),
          full(deck_b), full(fc1_W), full(fc1_b), full(fc2_W),
          full(fc2_b), full(out_W), full(out_b),
      ],
      out_specs=pl.BlockSpec((bb, 2), lambda i: (i, 0)),
      out_shape=jax.ShapeDtypeStruct((BATCH, 2), jnp.float32),
      interpret=interpret,
  )(x, hand_sum, deck_sum, sel, hand_W, hand_b, deck_W, deck_b,
    fc1_W, fc1_b, fc2_W, fc2_b, out_W, out_b)


def _count_selector():
  sel = jnp.zeros((2, XCOL), jnp.float32)
  sel = sel.at[0, 1:1 + MAX_HAND].set(1.0)
  sel = sel.at[1, 1 + MAX_HAND:].set(1.0)
  return sel


def kernel(x, table, hand_W, hand_b, deck_W, deck_b, fc1_W, fc1_b,
           fc2_W, fc2_b, out_W, out_b):
  hand_sum, deck_sum = _sc_pool(table, x)
  return _tc_mlp(x, hand_sum, deck_sum, _count_selector(), hand_W, hand_b,
                 deck_W, deck_b, fc1_W, fc1_b, fc2_W, fc2_b, out_W, out_b)


# speedup vs baseline: 29.1174x; 1.1967x over previous
import functools

import jax
import jax.numpy as jnp
from jax import lax
from jax.experimental import pallas as pl
from jax.experimental.pallas import tpu as pltpu
from jax.experimental.pallas import tpu_sc as plsc

VOCAB = 65536
EMBED = 32
BATCH = 16384
MAX_HAND = 7
MAX_DECK = 60
NCOL = MAX_HAND + MAX_DECK
XCOL = 1 + NCOL

NC = 2
NS = 16
NW = NC * NS
BPW = BATCH // NW
CHUNK = 128
NQ0 = BPW // CHUNK
NCH = NQ0 * XCOL
NGATH = NQ0 * NCOL


def _sc_pool(table, idx3):
  mesh = plsc.VectorSubcoreMesh(core_axis_name="c", subcore_axis_name="s")

  @functools.partial(
      pl.kernel,
      out_type=[
          jax.ShapeDtypeStruct((BATCH, EMBED), jnp.float32),
          jax.ShapeDtypeStruct((BATCH, EMBED), jnp.float32),
      ],
      mesh=mesh,
      compiler_params=pltpu.CompilerParams(use_tc_tiling_on_sc=False),
      scratch_types=[
          pltpu.VMEM((NCH, CHUNK), jnp.int32),
          pltpu.VMEM((BPW, EMBED), jnp.float32),
          pltpu.VMEM((BPW, EMBED), jnp.float32),
          pltpu.SemaphoreType.DMA,
          pltpu.SemaphoreType.DMA,
      ],
  )
  def sc_kernel(table_hbm, idx_hbm, hand_hbm, deck_hbm,
                idx_v, hacc, dacc, gsem, osem):
    wid = lax.axis_index("s") * NC + lax.axis_index("c")
    pltpu.sync_copy(idx_hbm.at[wid], idx_v)

    zv = jnp.zeros((16,), jnp.float32)

    def zero_row(i, carry):
      hacc[i, 0:16] = zv
      hacc[i, 16:EMBED] = zv
      dacc[i, 0:16] = zv
      dacc[i, 16:EMBED] = zv
      return carry

    lax.fori_loop(0, BPW, zero_row, 0)

    for q in range(NQ0):
      hdst = hacc.at[pl.ds(q * CHUNK, CHUNK)]
      ddst = dacc.at[pl.ds(q * CHUNK, CHUNK)]

      def hand_stream(jj, carry, q=q, hdst=hdst):
        pltpu.async_copy(table_hbm.at[idx_v.at[q * XCOL + jj]], hdst, gsem,
                         add=True)
        return carry

      def deck_stream(jj, carry, q=q, ddst=ddst):
        pltpu.async_copy(table_hbm.at[idx_v.at[q * XCOL + jj]], ddst, gsem,
                         add=True)
        return carry

      lax.fori_loop(1, 1 + MAX_HAND, hand_stream, 0)
      lax.fori_loop(1 + MAX_HAND, XCOL, deck_stream, 0)

    def drain(i, carry):
      pltpu.make_async_copy(
          table_hbm.at[pl.ds(0, CHUNK)],
          hacc.at[pl.ds(0, CHUNK)], gsem).wait()
      return carry

    lax.fori_loop(0, NGATH, drain, 0)

    out_base = wid * BPW
    cp_h = pltpu.async_copy(hacc, hand_hbm.at[pl.ds(out_base, BPW)], osem)
    cp_d = pltpu.async_copy(dacc, deck_hbm.at[pl.ds(out_base, BPW)], osem)
    cp_h.wait()
    cp_d.wait()

  return sc_kernel(table, idx3)


def _fmt_body(x_ref, idx_ref):
  xi = x_ref[...].astype(jnp.int32)
  for q in range(NQ0):
    idx_ref[0, q * XCOL:(q + 1) * XCOL, :] = (
        xi[q * CHUNK:(q + 1) * CHUNK, :].T)


def _format_idx(x, interpret=False):
  return pl.pallas_call(
      _fmt_body,
      grid=(NW,),
      in_specs=[pl.BlockSpec((BPW, XCOL), lambda i: (i, 0))],
      out_specs=pl.BlockSpec((1, NCH, CHUNK), lambda i: (i, 0, 0)),
      out_shape=jax.ShapeDtypeStruct((NW, NCH, CHUNK), jnp.int32),
      interpret=interpret,
  )(x)


def _dg(a, w, precision):
  return lax.dot_general(a, w, (((1,), (1,)), ((), ())),
                         precision=precision,
                         preferred_element_type=jnp.float32)


def _bf(a):
  return a.astype(jnp.bfloat16)


def _dgbf(a, w):
  return lax.dot_general(_bf(a), _bf(w), (((1,), (1,)), ((), ())),
                         preferred_element_type=jnp.float32)


def _tc_mlp_body(x_ref, hs_ref, ds_ref, sel_ref, hW_ref, hb_ref, dW_ref,
                 db_ref, f1W_ref, f1b_ref, f2W_ref, f2b_ref, oW_ref,
                 ob_ref, out_ref):
  xb = x_ref[...]
  mull = xb[:, 0:1]
  mask = (xb != 0.0).astype(jnp.float32)
  cnt = _dgbf(mask, sel_ref[...])
  hp = hs_ref[...] / (cnt[:, 0:1] + 1e-8)
  dp = ds_ref[...] / (cnt[:, 1:2] + 1e-8)
  hf = jnp.maximum(_dgbf(hp, hW_ref[...]) + hb_ref[...][None, :], 0.0)
  df = jnp.maximum(_dgbf(dp, dW_ref[...]) + db_ref[...][None, :], 0.0)
  comb = jnp.concatenate([mull, hf, df], axis=1)
  h1 = jnp.maximum(_dgbf(comb, f1W_ref[...]) + f1b_ref[...][None, :], 0.0)
  h2 = jnp.maximum(_dgbf(h1, f2W_ref[...]) + f2b_ref[...][None, :], 0.0)
  out_ref[...] = _dgbf(h2, oW_ref[...]) + ob_ref[...][None, :]


def _tc_mlp(x, hand_sum, deck_sum, sel, hand_W, hand_b, deck_W, deck_b,
            fc1_W, fc1_b, fc2_W, fc2_b, out_W, out_b, interpret=False):
  bb = 4096
  grid = (BATCH // bb,)
  full = lambda a: pl.BlockSpec(a.shape, lambda i: (0,) * a.ndim)
  return pl.pallas_call(
      _tc_mlp_body,
      grid=grid,
      in_specs=[
          pl.BlockSpec((bb, XCOL), lambda i: (i, 0)),
          pl.BlockSpec((bb, EMBED), lambda i: (i, 0)),
          pl.BlockSpec((bb, EMBED), lambda i: (i, 0)),
          full(sel), full(hand_W), full(hand_b), full(deck_W),
          full(deck_b), full(fc1_W), full(fc1_b), full(fc2_W),
          full(fc2_b), full(out_W), full(out_b),
      ],
      out_specs=pl.BlockSpec((bb, 2), lambda i: (i, 0)),
      out_shape=jax.ShapeDtypeStruct((BATCH, 2), jnp.float32),
      interpret=interpret,
  )(x, hand_sum, deck_sum, sel, hand_W, hand_b, deck_W, deck_b,
    fc1_W, fc1_b, fc2_W, fc2_b, out_W, out_b)


def _count_selector():
  sel = jnp.zeros((2, XCOL), jnp.float32)
  sel = sel.at[0, 1:1 + MAX_HAND].set(1.0)
  sel = sel.at[1, 1 + MAX_HAND:].set(1.0)
  return sel


def kernel(x, table, hand_W, hand_b, deck_W, deck_b, fc1_W, fc1_b,
           fc2_W, fc2_b, out_W, out_b):
  idx3 = _format_idx(x)
  hand_sum, deck_sum = _sc_pool(table, idx3)
  return _tc_mlp(x, hand_sum, deck_sum, _count_selector(), hand_W, hand_b,
                 deck_W, deck_b, fc1_W, fc1_b, fc2_W, fc2_b, out_W, out_b)
